# baseline (speedup 1.0000x reference)
import functools

import jax
import jax.numpy as jnp
from jax import lax
from jax.experimental import pallas as pl
from jax.experimental.pallas import tpu as pltpu
from jax.experimental.pallas import tpu_sc as plsc

_B = 4096
_D = 128
_NB = 2
_NW = 32
_CR = 256
_RSC = 24576
_STRIDE = _RSC // _NW
_NCH = _STRIDE // _CR
_MBLK = 24
_BR = 8192
_PBLK = 16


def _sc_body(te_in, ti_in, tip_in, idx_in, z_in, step_in,
             te_out, ti_out, tip_out,
             buf0, buf1, buf2, ibuf, ibuf2, svec, rsem, wsem, ssem, ssem2):
    wid = lax.axis_index("s") * 2 + lax.axis_index("c")

    pltpu.sync_copy(step_in, svec)
    s = jnp.max(svec[...])
    ps = lax.rem(s, jnp.int32(16))

    bufs = (buf0, buf1, buf2)
    nb = len(bufs)

    def ti_rd(dst):
        @pl.when(wid != s)
        def _():
            pltpu.make_async_copy(
                ti_in.at[pl.ds(wid * _B, _B)], dst, ssem).start()

        @pl.when(wid == s)
        def _():
            pltpu.make_async_copy(idx_in, dst, ssem).start()

    def tip_rd(dst):
        @pl.when(wid != ps)
        def _():
            pltpu.make_async_copy(
                tip_in.at[pl.ds(wid * _B, _B)], dst, ssem2).start()

        @pl.when(wid == ps)
        def _():
            pltpu.make_async_copy(idx_in, dst, ssem2).start()

    @pl.when(wid < _MBLK)
    def _():
        ti_rd(ibuf)

    @pl.when(wid < 16)
    def _():
        tip_rd(ibuf2)
    base = wid * _STRIDE
    s16 = s * (_B // _CR)

    def rd_start(c):
        g = wid * _NCH + c

        @pl.when(jnp.logical_or(g < s16, g >= s16 + (_B // _CR)))
        def _():
            pltpu.make_async_copy(
                te_in.at[pl.ds(base + c * _CR, _CR), :], bufs[c % nb], rsem
            ).start()

        @pl.when(jnp.logical_and(g >= s16, g < s16 + (_B // _CR)))
        def _():
            pltpu.make_async_copy(
                z_in.at[pl.ds((g - s16) * _CR, _CR), :], bufs[c % nb], rsem
            ).start()

    def rd_wait(c):
        pltpu.make_async_copy(
            z_in.at[pl.ds(0, _CR), :], bufs[c % nb], rsem).wait()

    def wr(c):
        return pltpu.make_async_copy(
            bufs[c % nb], te_out.at[pl.ds(base + c * _CR, _CR), :], wsem)

    rd_start(0)
    rd_start(1)
    for c in range(_NCH):
        if c >= 2:
            wr(c - 2).wait()
        if c + 1 < _NCH and c >= 1:
            rd_start(c + 1)
        rd_wait(c)
        wr(c).start()
    wr(_NCH - 2).wait()
    wr(_NCH - 1).wait()

    @pl.when(wid < _MBLK)
    def _():
        pltpu.make_async_copy(idx_in, ibuf, ssem).wait()
        pltpu.make_async_copy(
            ibuf, ti_out.at[pl.ds(wid * _B, _B)], ssem).start()

    @pl.when(wid < 16)
    def _():
        pltpu.make_async_copy(idx_in, ibuf2, ssem2).wait()
        pltpu.make_async_copy(
            ibuf2, tip_out.at[pl.ds(wid * _B, _B)], ssem2).start()

    @pl.when(wid < _MBLK)
    def _():
        pltpu.make_async_copy(
            idx_in, ti_out.at[pl.ds(wid * _B, _B)], ssem).wait()

    @pl.when(wid < 16)
    def _():
        pltpu.make_async_copy(
            idx_in, tip_out.at[pl.ds(wid * _B, _B)], ssem2).wait()


def _te_tail_body(step_ref, te_alias, te_in, z_in, te_out):
    i = pl.program_id(0)
    s = step_ref[0]
    j = i + _RSC // _BR
    sblk, soff = s // 2, lax.rem(s, 2) * _B

    te_out[...] = te_in[...]

    @pl.when(j == sblk)
    def _():
        te_out[pl.ds(soff, _B), :] = z_in[...]


def _tc_finish_ref(te_sc, train_embeddings_ref, Z_ssps, step):
    M = train_embeddings_ref.shape[0]
    b0 = _RSC // _BR
    nblk = M // _BR - b0

    return pl.pallas_call(
        _te_tail_body,
        grid_spec=pltpu.PrefetchScalarGridSpec(
            num_scalar_prefetch=1,
            grid=(nblk,),
            in_specs=[
                pl.BlockSpec(memory_space=pltpu.MemorySpace.HBM),
                pl.BlockSpec((_BR, _D), lambda i, s: (i + b0, 0)),
                pl.BlockSpec((_B, _D), lambda i, s: (0, 0)),
            ],
            out_specs=pl.BlockSpec((_BR, _D), lambda i, s: (i + b0, 0)),
        ),
        out_shape=jax.ShapeDtypeStruct((M, _D), jnp.float32),
        input_output_aliases={1: 0},
        compiler_params=pltpu.CompilerParams(
            dimension_semantics=("arbitrary",),
        ),
    )(step, te_sc, train_embeddings_ref, Z_ssps)


def _tc_body(step_ref, tep_in, emb_in, tep_out):
    i = pl.program_id(0)
    s = step_ref[0]
    ps = lax.rem(s, jnp.int32(16))
    j = lax.rem(i, _PBLK // _NB)
    pblk, poff = ps // 2, lax.rem(ps, 2) * _B

    tep_out[...] = tep_in[...]

    @pl.when(j == pblk)
    def _():
        tep_out[pl.ds(poff, _B), :] = emb_in[0]


def _tc_update_pos(train_embeddings_pos, embeddings, step):
    P = train_embeddings_pos.shape[1]
    tep_flat = train_embeddings_pos.reshape(_NB * P, _D)

    out = pl.pallas_call(
        _tc_body,
        grid_spec=pltpu.PrefetchScalarGridSpec(
            num_scalar_prefetch=1,
            grid=(_PBLK,),
            in_specs=[
                pl.BlockSpec((_BR, _D), lambda i, s: (i, 0)),
                pl.BlockSpec((1, _B, _D),
                             lambda i, s: (i // (_PBLK // _NB), 0, 0)),
            ],
            out_specs=pl.BlockSpec((_BR, _D), lambda i, s: (i, 0)),
        ),
        out_shape=jax.ShapeDtypeStruct((_NB * P, _D), jnp.float32),
        compiler_params=pltpu.CompilerParams(
            dimension_semantics=("arbitrary",),
        ),
    )(step, tep_flat, embeddings)
    return out.reshape(_NB, P, _D)


def kernel(train_indices_ref, train_embeddings_ref, train_indices_pos,
           train_embeddings_pos, indices, Z_ssps, embeddings, step_rel):
    M = train_embeddings_ref.shape[0]
    P = train_indices_pos.shape[0]
    step = jnp.asarray(step_rel, jnp.int32)
    step_vec = jnp.full((16,), step, jnp.int32)

    sc = functools.partial(
        pl.kernel,
        out_type=[
            jax.ShapeDtypeStruct((M, _D), jnp.float32),
            jax.ShapeDtypeStruct((M,), jnp.int32),
            jax.ShapeDtypeStruct((P,), jnp.int32),
        ],
        mesh=plsc.VectorSubcoreMesh(core_axis_name="c", subcore_axis_name="s"),
        scratch_types=[
            pltpu.VMEM((_CR, _D), jnp.float32),
            pltpu.VMEM((_CR, _D), jnp.float32),
            pltpu.VMEM((_CR, _D), jnp.float32),
            pltpu.VMEM((_B,), jnp.int32),
            pltpu.VMEM((_B,), jnp.int32),
            pltpu.VMEM((16,), jnp.int32),
            pltpu.SemaphoreType.DMA,
            pltpu.SemaphoreType.DMA,
            pltpu.SemaphoreType.DMA,
            pltpu.SemaphoreType.DMA,
        ],
        compiler_params=pltpu.CompilerParams(needs_layout_passes=False),
    )(_sc_body)

    te_sc, ti_out, tip_out = sc(
        train_embeddings_ref, train_indices_ref, train_indices_pos, indices,
        Z_ssps, step_vec)

    tep_out = _tc_update_pos(train_embeddings_pos, embeddings,
                             step.reshape(1))

    te_out = _tc_finish_ref(te_sc, train_embeddings_ref, Z_ssps,
                            step.reshape(1))

    return (ti_out, te_out, tip_out, tep_out)


# speedup vs baseline: 1.0165x; 1.0165x over previous
import functools

import jax
import jax.numpy as jnp
from jax import lax
from jax.experimental import pallas as pl
from jax.experimental.pallas import tpu as pltpu
from jax.experimental.pallas import tpu_sc as plsc

_B = 4096
_D = 128
_NB = 2
_BR = 8192
_MBLK = 12
_PBLK = 8
_GRID = _NB * _PBLK


def _sc_body(ti_in, tip_in, idx_in, step_in,
             ti_out, tip_out,
             ibuf, ibuf2, svec, ssem, ssem2):
    wid = lax.axis_index("s") * 2 + lax.axis_index("c")

    pltpu.sync_copy(step_in, svec)
    s = jnp.max(svec[...])
    ps = lax.rem(s, jnp.int32(16))

    @pl.when(wid < _MBLK * 2)
    def _():
        @pl.when(wid != s)
        def _():
            pltpu.make_async_copy(
                ti_in.at[pl.ds(wid * _B, _B)], ibuf, ssem).start()

        @pl.when(wid == s)
        def _():
            pltpu.make_async_copy(idx_in, ibuf, ssem).start()

    @pl.when(wid < 16)
    def _():
        @pl.when(wid != ps)
        def _():
            pltpu.make_async_copy(
                tip_in.at[pl.ds(wid * _B, _B)], ibuf2, ssem2).start()

        @pl.when(wid == ps)
        def _():
            pltpu.make_async_copy(idx_in, ibuf2, ssem2).start()

    @pl.when(wid < _MBLK * 2)
    def _():
        pltpu.make_async_copy(idx_in, ibuf, ssem).wait()
        pltpu.make_async_copy(
            ibuf, ti_out.at[pl.ds(wid * _B, _B)], ssem).start()

    @pl.when(wid < 16)
    def _():
        pltpu.make_async_copy(idx_in, ibuf2, ssem2).wait()
        pltpu.make_async_copy(
            ibuf2, tip_out.at[pl.ds(wid * _B, _B)], ssem2).start()

    @pl.when(wid < _MBLK * 2)
    def _():
        pltpu.make_async_copy(
            idx_in, ti_out.at[pl.ds(wid * _B, _B)], ssem).wait()

    @pl.when(wid < 16)
    def _():
        pltpu.make_async_copy(
            idx_in, tip_out.at[pl.ds(wid * _B, _B)], ssem2).wait()


def _tc_body(step_ref, te_in, tep_in, z_in, emb_in, te_out, tep_out):
    i = pl.program_id(0)
    s = step_ref[0]
    sblk, soff = s // 2, lax.rem(s, 2) * _B
    ps = lax.rem(s, jnp.int32(16))
    pblk, poff = ps // 2, lax.rem(ps, 2) * _B
    j = lax.rem(i, _PBLK)

    @pl.when(i < _MBLK)
    def _():
        te_out[...] = te_in[...]

        @pl.when(i == sblk)
        def _():
            te_out[pl.ds(soff, _B), :] = z_in[...]

    tep_out[...] = tep_in[...]

    @pl.when(j == pblk)
    def _():
        tep_out[0, pl.ds(poff, _B), :] = emb_in[0]


def kernel(train_indices_ref, train_embeddings_ref, train_indices_pos,
           train_embeddings_pos, indices, Z_ssps, embeddings, step_rel):
    M = train_embeddings_ref.shape[0]
    P = train_indices_pos.shape[0]
    step = jnp.asarray(step_rel, jnp.int32)
    step_vec = jnp.full((16,), step, jnp.int32)

    sc = functools.partial(
        pl.kernel,
        out_type=[
            jax.ShapeDtypeStruct((M,), jnp.int32),
            jax.ShapeDtypeStruct((P,), jnp.int32),
        ],
        mesh=plsc.VectorSubcoreMesh(core_axis_name="c", subcore_axis_name="s"),
        scratch_types=[
            pltpu.VMEM((_B,), jnp.int32),
            pltpu.VMEM((_B,), jnp.int32),
            pltpu.VMEM((16,), jnp.int32),
            pltpu.SemaphoreType.DMA,
            pltpu.SemaphoreType.DMA,
        ],
        compiler_params=pltpu.CompilerParams(needs_layout_passes=False),
    )(_sc_body)

    ti_out, tip_out = sc(train_indices_ref, train_indices_pos, indices,
                         step_vec)

    te_out, tep_out = pl.pallas_call(
        _tc_body,
        grid_spec=pltpu.PrefetchScalarGridSpec(
            num_scalar_prefetch=1,
            grid=(_GRID,),
            in_specs=[
                pl.BlockSpec((_BR, _D),
                             lambda i, s: (jnp.minimum(i, _MBLK - 1), 0)),
                pl.BlockSpec((1, _BR, _D),
                             lambda i, s: (i // _PBLK, lax.rem(i, _PBLK), 0)),
                pl.BlockSpec((_B, _D), lambda i, s: (0, 0)),
                pl.BlockSpec((1, _B, _D), lambda i, s: (i // _PBLK, 0, 0)),
            ],
            out_specs=[
                pl.BlockSpec((_BR, _D),
                             lambda i, s: (jnp.minimum(i, _MBLK - 1), 0)),
                pl.BlockSpec((1, _BR, _D),
                             lambda i, s: (i // _PBLK, lax.rem(i, _PBLK), 0)),
            ],
        ),
        out_shape=[
            jax.ShapeDtypeStruct((M, _D), jnp.float32),
            jax.ShapeDtypeStruct((_NB, P, _D), jnp.float32),
        ],
        compiler_params=pltpu.CompilerParams(
            dimension_semantics=("arbitrary",),
        ),
    )(step.reshape(1), train_embeddings_ref, train_embeddings_pos, Z_ssps,
      embeddings)

    return (ti_out, te_out, tip_out, tep_out)


# speedup vs baseline: 1.0491x; 1.0321x over previous
import functools

import jax
import jax.numpy as jnp
from jax import lax
from jax.experimental import pallas as pl
from jax.experimental.pallas import tpu as pltpu
from jax.experimental.pallas import tpu_sc as plsc

_B = 4096
_D = 128
_NB = 2
_NW = 32
_CR = 256
_STRIDE = 3072
_NCH = 12
_MBLK = 24
_BR = 8192
_PBLK = 16


def _sc_body(te_in, ti_in, tip_in, idx_in, z_in, step_in,
             te_out, ti_out, tip_out,
             shared, ibuf, ibuf2, svec, rsem, wsem, ssem, ssem2):
    sid = lax.axis_index("s")
    wid = sid * 2 + lax.axis_index("c")

    pltpu.sync_copy(step_in, svec)
    s = jnp.max(svec[...])
    ps = lax.rem(s, jnp.int32(16))

    nb = 3
    bufs = tuple(shared.at[sid, j] for j in range(nb))

    def ti_rd(dst):
        @pl.when(wid != s)
        def _():
            pltpu.make_async_copy(
                ti_in.at[pl.ds(wid * _B, _B)], dst, ssem).start()

        @pl.when(wid == s)
        def _():
            pltpu.make_async_copy(idx_in, dst, ssem).start()

    def tip_rd(dst):
        @pl.when(wid != ps)
        def _():
            pltpu.make_async_copy(
                tip_in.at[pl.ds(wid * _B, _B)], dst, ssem2).start()

        @pl.when(wid == ps)
        def _():
            pltpu.make_async_copy(idx_in, dst, ssem2).start()

    @pl.when(wid < _MBLK)
    def _():
        ti_rd(ibuf)

    @pl.when(wid < 16)
    def _():
        tip_rd(ibuf2)
    base = wid * _STRIDE
    s16 = s * (_B // _CR)

    def rd_start(c):
        g = wid * _NCH + c

        @pl.when(jnp.logical_or(g < s16, g >= s16 + (_B // _CR)))
        def _():
            pltpu.make_async_copy(
                te_in.at[pl.ds(base + c * _CR, _CR), :], bufs[c % nb], rsem
            ).start()

        @pl.when(jnp.logical_and(g >= s16, g < s16 + (_B // _CR)))
        def _():
            pltpu.make_async_copy(
                z_in.at[pl.ds((g - s16) * _CR, _CR), :], bufs[c % nb], rsem
            ).start()

    def rd_wait(c):
        pltpu.make_async_copy(
            z_in.at[pl.ds(0, _CR), :], bufs[c % nb], rsem).wait()

    def wr(c):
        return pltpu.make_async_copy(
            bufs[c % nb], te_out.at[pl.ds(base + c * _CR, _CR), :], wsem)

    rd_start(0)
    rd_start(1)
    for c in range(_NCH):
        if c >= 2:
            wr(c - 2).wait()
        if c + 1 < _NCH and c >= 1:
            rd_start(c + 1)
        rd_wait(c)
        wr(c).start()
    wr(_NCH - 2).wait()
    wr(_NCH - 1).wait()

    @pl.when(wid < _MBLK)
    def _():
        pltpu.make_async_copy(idx_in, ibuf, ssem).wait()
        pltpu.make_async_copy(
            ibuf, ti_out.at[pl.ds(wid * _B, _B)], ssem).start()

    @pl.when(wid < 16)
    def _():
        pltpu.make_async_copy(idx_in, ibuf2, ssem2).wait()
        pltpu.make_async_copy(
            ibuf2, tip_out.at[pl.ds(wid * _B, _B)], ssem2).start()

    @pl.when(wid < _MBLK)
    def _():
        pltpu.make_async_copy(
            idx_in, ti_out.at[pl.ds(wid * _B, _B)], ssem).wait()

    @pl.when(wid < 16)
    def _():
        pltpu.make_async_copy(
            idx_in, tip_out.at[pl.ds(wid * _B, _B)], ssem2).wait()


def _tc_body(step_ref, tep_in, emb_in, tep_out):
    i = pl.program_id(0)
    s = step_ref[0]
    ps = lax.rem(s, jnp.int32(16))
    j = lax.rem(i, _PBLK // _NB)
    pblk, poff = ps // 2, lax.rem(ps, 2) * _B

    tep_out[...] = tep_in[...]

    @pl.when(j == pblk)
    def _():
        tep_out[pl.ds(poff, _B), :] = emb_in[0]


def _tc_update_pos(train_embeddings_pos, embeddings, step):
    P = train_embeddings_pos.shape[1]
    tep_flat = train_embeddings_pos.reshape(_NB * P, _D)

    out = pl.pallas_call(
        _tc_body,
        grid_spec=pltpu.PrefetchScalarGridSpec(
            num_scalar_prefetch=1,
            grid=(_PBLK,),
            in_specs=[
                pl.BlockSpec((_BR, _D), lambda i, s: (i, 0)),
                pl.BlockSpec((1, _B, _D),
                             lambda i, s: (i // (_PBLK // _NB), 0, 0)),
            ],
            out_specs=pl.BlockSpec((_BR, _D), lambda i, s: (i, 0)),
        ),
        out_shape=jax.ShapeDtypeStruct((_NB * P, _D), jnp.float32),
        compiler_params=pltpu.CompilerParams(
            dimension_semantics=("arbitrary",),
        ),
    )(step, tep_flat, embeddings)
    return out.reshape(_NB, P, _D)


def kernel(train_indices_ref, train_embeddings_ref, train_indices_pos,
           train_embeddings_pos, indices, Z_ssps, embeddings, step_rel):
    M = train_embeddings_ref.shape[0]
    P = train_indices_pos.shape[0]
    step = jnp.asarray(step_rel, jnp.int32)
    step_vec = jnp.full((16,), step, jnp.int32)

    sc = functools.partial(
        pl.kernel,
        out_type=[
            jax.ShapeDtypeStruct((M, _D), jnp.float32),
            jax.ShapeDtypeStruct((M,), jnp.int32),
            jax.ShapeDtypeStruct((P,), jnp.int32),
        ],
        mesh=plsc.VectorSubcoreMesh(core_axis_name="c", subcore_axis_name="s"),
        scratch_types=[
            pltpu.MemorySpace.VMEM_SHARED((16, 3, _CR, _D), jnp.float32),
            pltpu.VMEM((_B,), jnp.int32),
            pltpu.VMEM((_B,), jnp.int32),
            pltpu.VMEM((16,), jnp.int32),
            pltpu.SemaphoreType.DMA,
            pltpu.SemaphoreType.DMA,
            pltpu.SemaphoreType.DMA,
            pltpu.SemaphoreType.DMA,
        ],
        compiler_params=pltpu.CompilerParams(needs_layout_passes=False),
    )(_sc_body)

    te_out, ti_out, tip_out = sc(
        train_embeddings_ref, train_indices_ref, train_indices_pos, indices,
        Z_ssps, step_vec)

    tep_out = _tc_update_pos(train_embeddings_pos, embeddings,
                             step.reshape(1))

    return (ti_out, te_out, tip_out, tep_out)


# speedup vs baseline: 1.2520x; 1.1933x over previous
import jax
import jax.numpy as jnp
from jax import lax
from jax.experimental import pallas as pl
from jax.experimental.pallas import tpu as pltpu

_B = 4096
_D = 128
_BR = 8192
_MBLK = 6
_PBLK = 8
_NB = 2
_GRID = _NB * _PBLK


def _body(step_ref,
          ti_ref_in, te_ref_in, tip_in, tep_in, idx2_in, z_in, emb_in,
          ti_ref_out, te_ref_out, tip_out, tep_out):
    i = pl.program_id(0)
    s = step_ref[0]
    sblk, soff = s // 4, lax.rem(s, 4) * _B
    ps = lax.rem(s, _NB * _PBLK)
    pblk, poff = ps // 2, lax.rem(ps, 2) * _B

    @pl.when(i < _MBLK)
    def _():
        te_ref_out[...] = te_ref_in[...]

        @pl.when(i == sblk)
        def _():
            te_ref_out[pl.ds(soff, _B), :] = z_in[...]

    j = lax.rem(i, _PBLK)
    tep_out[...] = tep_in[...]

    @pl.when(j == pblk)
    def _():
        tep_out[0, pl.ds(poff, _B), :] = emb_in[0]

    @pl.when(i == 0)
    def _():
        rows = _B // _D
        ti_ref_out[...] = ti_ref_in[...]
        ti_ref_out[pl.ds(s * rows, rows), :] = idx2_in[...]
        tip_out[...] = tip_in[...]
        tip_out[pl.ds(lax.rem(s * rows, tip_out.shape[0]), rows), :] = idx2_in[...]


def kernel(train_indices_ref, train_embeddings_ref, train_indices_pos,
           train_embeddings_pos, indices, Z_ssps, embeddings, step_rel):
    M = train_embeddings_ref.shape[0]
    P = train_indices_pos.shape[0]
    step = jnp.asarray(step_rel, jnp.int32).reshape(1)

    ti2 = train_indices_ref.reshape(M // _D, _D)
    tip2 = train_indices_pos.reshape(P // _D, _D)
    idx2 = indices.reshape(_B // _D, _D)

    full = lambda shape: pl.BlockSpec(shape, lambda i, s: (0,) * len(shape))

    out = pl.pallas_call(
        _body,
        grid_spec=pltpu.PrefetchScalarGridSpec(
            num_scalar_prefetch=1,
            grid=(_GRID,),
            in_specs=[
                full(ti2.shape),
                pl.BlockSpec((2 * _BR, _D),
                             lambda i, s: (jnp.minimum(i, _MBLK - 1), 0)),
                full(tip2.shape),
                pl.BlockSpec((1, _BR, _D),
                             lambda i, s: (i // _PBLK, lax.rem(i, _PBLK), 0)),
                full(idx2.shape),
                full((_B, _D)),
                pl.BlockSpec((1, _B, _D), lambda i, s: (i // _PBLK, 0, 0)),
            ],
            out_specs=[
                full(ti2.shape),
                pl.BlockSpec((2 * _BR, _D),
                             lambda i, s: (jnp.minimum(i, _MBLK - 1), 0)),
                full(tip2.shape),
                pl.BlockSpec((1, _BR, _D),
                             lambda i, s: (i // _PBLK, lax.rem(i, _PBLK), 0)),
            ],
        ),
        out_shape=[
            jax.ShapeDtypeStruct(ti2.shape, jnp.int32),
            jax.ShapeDtypeStruct((M, _D), jnp.float32),
            jax.ShapeDtypeStruct(tip2.shape, jnp.int32),
            jax.ShapeDtypeStruct((_NB, P, _D), jnp.float32),
        ],
        compiler_params=pltpu.CompilerParams(
            dimension_semantics=("arbitrary",),
        ),
    )(step, ti2, train_embeddings_ref, tip2, train_embeddings_pos, idx2,
      Z_ssps, embeddings)

    return (out[0].reshape(M), out[1], out[2].reshape(P), out[3])
